# Initial kernel scaffold; baseline (speedup 1.0000x reference)
import jax
import jax.numpy as jnp
from jax.experimental import pallas as pl


def kernel(x, table, W, b):
    raise NotImplementedError("write your pallas kernel here")


# speedup vs baseline: 8.1172x; 8.1172x over previous
import dataclasses
import functools

import jax
import jax.numpy as jnp
from jax import lax
from jax.experimental import pallas as pl
from jax.experimental.pallas import tpu as pltpu
from jax.experimental.pallas import tpu_sc as plsc

V = 1000000
E = 16
B = 16384
L = 200

PROJ_ROWS = V * E // 128
PROJ_BLK = 5000
GROUPS = 128 // E

NC, NS = 2, 16
NW = NC * NS
RPW = B // NW
CH_ROWS = 64
NCH = RPW // CH_ROWS
CHI = CH_ROWS * L


def _proj_body(t_ref, m_ref, o_ref):
    o_ref[...] = jnp.dot(t_ref[...], m_ref[...],
                         preferred_element_type=jnp.float32)


def _project_table(table, W):
    t_r = table.reshape(PROJ_ROWS, 128)
    m = jnp.kron(jnp.eye(GROUPS, dtype=jnp.float32), W.reshape(E, 1))
    tv2 = pl.pallas_call(
        _proj_body,
        grid=(PROJ_ROWS // PROJ_BLK,),
        in_specs=[
            pl.BlockSpec((PROJ_BLK, 128), lambda i: (i, 0)),
            pl.BlockSpec((128, GROUPS), lambda i: (0, 0)),
        ],
        out_specs=pl.BlockSpec((PROJ_BLK, GROUPS), lambda i: (i, 0)),
        out_shape=jax.ShapeDtypeStruct((PROJ_ROWS, GROUPS), jnp.float32),
    )(t_r, m)
    return tv2.reshape(V)


def _sc_body(xflat_hbm, tv_hbm, bvec_hbm, out_hbm, idx_v, g_v, out_v, b_v, sem):
    wid = lax.axis_index("s") * NC + lax.axis_index("c")
    base = wid * RPW * L
    pltpu.sync_copy(bvec_hbm, b_v)
    for c in range(NCH):
        pltpu.sync_copy(xflat_hbm.at[pl.ds(base + c * CHI, CHI)], idx_v)
        pltpu.async_copy(tv_hbm.at[idx_v], g_v, sem).wait()
        for grp in range(CH_ROWS // 16):
            row_starts = (lax.iota(jnp.int32, 16) + grp * 16) * L

            def red_body(i, acc, row_starts=row_starts):
                return acc + plsc.load_gather(g_v, [row_starts + i])

            acc = lax.fori_loop(0, L, red_body,
                                jnp.zeros((16,), jnp.float32))
            z = acc * jnp.float32(1.0 / L) + b_v[...]
            out_v[pl.ds(c * CH_ROWS + grp * 16, 16)] = (
                jnp.float32(1.0) / (jnp.float32(1.0) + jnp.exp(-z)))
    pltpu.sync_copy(out_v, out_hbm.at[pl.ds(wid * RPW, RPW)])


def kernel(x, table, W, b):
    tv = _project_table(table, W)
    xflat = x.reshape(B * L).astype(jnp.int32)
    bvec = jnp.broadcast_to(b.astype(jnp.float32), (16,))
    cp = pltpu.CompilerParams()
    if "needs_layout_passes" in pltpu.CompilerParams.__dataclass_fields__:
        cp = dataclasses.replace(cp, needs_layout_passes=False)
    sc = functools.partial(
        pl.kernel,
        compiler_params=cp,
        out_type=jax.ShapeDtypeStruct((B,), jnp.float32),
        mesh=plsc.VectorSubcoreMesh(core_axis_name="c", subcore_axis_name="s"),
        scratch_types=[
            pltpu.VMEM((CHI,), jnp.int32),
            pltpu.VMEM((CHI,), jnp.float32),
            pltpu.VMEM((RPW,), jnp.float32),
            pltpu.VMEM((16,), jnp.float32),
            pltpu.SemaphoreType.DMA,
        ],
    )(_sc_body)
    out = sc(xflat, tv, bvec)
    return out.reshape(B, 1)
